# baseline (speedup 1.0000x reference)
import functools

import jax
import jax.numpy as jnp
import numpy as np
from jax import lax
from jax.experimental import pallas as pl
from jax.experimental.pallas import tpu as pltpu
from jax.experimental.pallas import tpu_sc as plsc

_N = 10000
_E = 320000
_D = 128

_NC = 2
_NS = 16
_CHUNK = 80
_CPC = _E // _CHUNK // _NC
_TPT = _CPC // _NS
_PAIRS = (_TPT - 1) // 2
_RPT = 624

_BE = 1280


@functools.partial(
    pl.kernel,
    out_type=jax.ShapeDtypeStruct((_NC, _N, _D), jnp.float32),
    mesh=plsc.VectorSubcoreMesh(core_axis_name="c", subcore_axis_name="s"),
    compiler_params=pltpu.CompilerParams(needs_layout_passes=False),
    scratch_types=[
        pltpu.VMEM((_CHUNK,), jnp.int32),
        pltpu.VMEM((_CHUNK,), jnp.int32),
        pltpu.VMEM((_CHUNK,), jnp.int32),
        pltpu.VMEM((_CHUNK,), jnp.int32),
        pltpu.VMEM((_CHUNK, _D), jnp.float32),
        pltpu.VMEM((_CHUNK, _D), jnp.float32),
        pltpu.VMEM((_CHUNK, _D), jnp.float32),
        pltpu.VMEM((_CHUNK, _D), jnp.float32),
        pltpu.SemaphoreType.DMA,
        pltpu.SemaphoreType.DMA,
        pltpu.SemaphoreType.DMA,
        pltpu.SemaphoreType.DMA,
        pltpu.SemaphoreType.DMA,
        pltpu.SemaphoreType.DMA,
        pltpu.VMEM_SHARED((_N, _D), jnp.float32),
    ],
)
def _sc_aggregate(h_hbm, ea_hbm, src_hbm, dst_hbm, out_hbm,
                  src0, src1, dst0, dst1, rows0, rows1, eab0, eab1,
                  si0, si1, sg0, sg1, se0, se1, acc):
    c = lax.axis_index("c")
    s = lax.axis_index("s")
    srcv = (src0, src1)
    dstv = (dst0, dst1)
    rows = (rows0, rows1)
    eab = (eab0, eab1)
    si = (si0, si1)
    sg = (sg0, sg1)
    se = (se0, se1)
    zero = jnp.zeros((16,), jnp.float32)

    def _zrow(r, carry):
        for j in range(8):
            rows0[r, pl.ds(j * 16, 16)] = zero
        return carry
    lax.fori_loop(0, _CHUNK, _zrow, 0)

    base = s * _RPT
    for j in range(_RPT // _CHUNK):
        pltpu.sync_copy(rows0, acc.at[pl.ds(base + j * _CHUNK, _CHUNK)])
    _REM = _RPT - (_RPT // _CHUNK) * _CHUNK
    pltpu.sync_copy(rows0.at[pl.ds(0, _REM)],
                    acc.at[pl.ds(base + _RPT - _REM, _REM)])

    @pl.when(s == _NS - 1)
    def _ztail():
        pltpu.sync_copy(rows0.at[pl.ds(0, _N - _NS * _RPT)],
                        acc.at[pl.ds(_NS * _RPT, _N - _NS * _RPT)])
    plsc.subcore_barrier()

    def _echunk(t):
        return (c * _CPC + t * _NS + s) * _CHUNK

    def _issue_idx(t, b):
        e0 = _echunk(t)
        pltpu.async_copy(src_hbm.at[pl.ds(e0, _CHUNK)], srcv[b], si[b])
        pltpu.async_copy(dst_hbm.at[pl.ds(e0, _CHUNK)], dstv[b], si[b])

    def _drain_idx(b):
        pltpu.make_async_copy(src_hbm.at[pl.ds(0, _CHUNK)], srcv[b], si[b]).wait()
        pltpu.make_async_copy(dst_hbm.at[pl.ds(0, _CHUNK)], dstv[b], si[b]).wait()

    def _issue_dat(t, b):
        pltpu.async_copy(h_hbm.at[srcv[b]], rows[b], sg[b])
        pltpu.async_copy(ea_hbm.at[pl.ds(_echunk(t), _CHUNK)], eab[b], se[b])

    def _drain_dat(b):
        pltpu.make_async_copy(h_hbm.at[pl.ds(0, _CHUNK)], rows[b], sg[b]).wait()
        pltpu.make_async_copy(ea_hbm.at[pl.ds(0, _CHUNK)], eab[b], se[b]).wait()

    def _compute(b):
        def _qrow(q, carry2):
            for k in range(8):
                sl = pl.ds(16 * k, 16)
                eab[b][q, sl] = jnp.maximum(rows[b][q, sl] + eab[b][q, sl], 0.0)
            return carry2
        lax.fori_loop(0, _CHUNK, _qrow, 0)

    _issue_idx(0, 0)
    _drain_idx(0)
    _issue_dat(0, 0)
    _issue_idx(1, 1)

    def _pair(j, carry):
        for b in (0, 1):
            t = 2 * j + b
            nb = 1 - b
            _drain_idx(nb)
            _issue_dat(t + 1, nb)
            _drain_dat(b)
            _compute(b)
            pltpu.sync_copy(eab[b], acc.at[dstv[b]], add=True)
            if b == 0:
                _issue_idx(t + 2, b)
            else:
                @pl.when(j < _PAIRS - 1)
                def _():
                    _issue_idx(t + 2, b)
        return carry
    lax.fori_loop(0, _PAIRS, _pair, 0)

    _drain_dat(0)
    _compute(0)
    pltpu.sync_copy(eab0, acc.at[dstv[0]], add=True)
    plsc.subcore_barrier()

    pltpu.sync_copy(acc.at[pl.ds(base, _RPT)],
                    out_hbm.at[c, pl.ds(base, _RPT)])

    @pl.when(s == _NS - 1)
    def _dtail():
        pltpu.sync_copy(acc.at[pl.ds(_NS * _RPT, _N - _NS * _RPT)],
                        out_hbm.at[c, pl.ds(_NS * _RPT, _N - _NS * _RPT)])


def _edge_mm_body(a_ref, w0_ref, b0_ref, w1_ref, b1_ref, o0_ref, o1_ref):
    a = a_ref[...]
    o0_ref[...] = jnp.dot(a, w0_ref[...], preferred_element_type=jnp.float32) + b0_ref[...]
    o1_ref[...] = jnp.dot(a, w1_ref[...], preferred_element_type=jnp.float32) + b1_ref[...]


def _edge_mm(edge_attr, w0t, b0, w1t, b1):
    grid = (_E // _BE,)
    blk = pl.BlockSpec((_BE, _D), lambda i: (i, 0))
    wblk = pl.BlockSpec((_D, _D), lambda i: (0, 0))
    bblk = pl.BlockSpec((1, _D), lambda i: (0, 0))
    return pl.pallas_call(
        _edge_mm_body,
        grid=grid,
        in_specs=[blk, wblk, bblk, wblk, bblk],
        out_specs=[blk, blk],
        out_shape=[jax.ShapeDtypeStruct((_E, _D), jnp.float32)] * 2,
    )(edge_attr, w0t, b0, w1t, b1)


def _bn(x, g, b):
    m = jnp.mean(x, axis=0, keepdims=True)
    xc = x - m
    v = jnp.mean(xc * xc, axis=0, keepdims=True)
    return xc * lax.rsqrt(v + 1e-5) * g + b


def _bn_in_body(x_ref, g_ref, b_ref, o_ref):
    o_ref[...] = _bn(x_ref[...], g_ref[...], b_ref[...])


def _bn_in(x, g, b):
    return pl.pallas_call(
        _bn_in_body,
        out_shape=jax.ShapeDtypeStruct((_N, _D), jnp.float32),
    )(x, g.reshape(1, _D), b.reshape(1, _D))


def _node0_body(h_ref, p0_ref, p1_ref, w_ref, b_ref, g_ref, bb_ref, o_ref):
    u = h_ref[...] + p0_ref[...] + p1_ref[...]
    t = jnp.tanh(jnp.dot(u, w_ref[...], preferred_element_type=jnp.float32) + b_ref[...])
    o_ref[...] = _bn(t, g_ref[...], bb_ref[...])


def _node0(h, p0, p1, wt, b, g, bb):
    return pl.pallas_call(
        _node0_body,
        out_shape=jax.ShapeDtypeStruct((_N, _D), jnp.float32),
    )(h, p0, p1, wt, b.reshape(1, _D), g.reshape(1, _D), bb.reshape(1, _D))


def _node1_body(h1_ref, p0_ref, p1_ref, w_ref, b_ref, g_ref, bb_ref, fc_ref, o_ref):
    h1 = h1_ref[...]
    u = h1 + p0_ref[...] + p1_ref[...]
    t = jnp.tanh(jnp.dot(u, w_ref[...], preferred_element_type=jnp.float32) + b_ref[...])
    h2 = _bn(t, g_ref[...], bb_ref[...])
    h3 = jnp.tanh(jnp.dot(h2, fc_ref[...], preferred_element_type=jnp.float32))
    o_ref[:, 0:_D] = h1
    o_ref[:, _D:2 * _D] = h2
    o_ref[:, 2 * _D:3 * _D] = h3


def _node1(h1, p0, p1, wt, b, g, bb, fct):
    return pl.pallas_call(
        _node1_body,
        out_shape=jax.ShapeDtypeStruct((_N, 3 * _D), jnp.float32),
    )(h1, p0, p1, wt, b.reshape(1, _D), g.reshape(1, _D), bb.reshape(1, _D), fct)


def kernel(x, edge_index, edge_attr, bn_in_g, bn_in_b,
           lin_e0_W, lin_e0_b, nn0_W, nn0_b, bn0_g, bn0_b,
           lin_e1_W, lin_e1_b, nn1_W, nn1_b, bn1_g, bn1_b,
           fc_W):
    ea0, ea1 = _edge_mm(edge_attr, lin_e0_W.T, lin_e0_b.reshape(1, _D),
                        lin_e1_W.T, lin_e1_b.reshape(1, _D))
    h = _bn_in(x, bn_in_g, bn_in_b)
    src = edge_index[0]
    dst = edge_index[1]

    parts0 = _sc_aggregate(h, ea0, src, dst)
    h1 = _node0(h, parts0[0], parts0[1], nn0_W.T, nn0_b, bn0_g, bn0_b)

    parts1 = _sc_aggregate(h1, ea1, src, dst)
    return _node1(h1, parts1[0], parts1[1], nn1_W.T, nn1_b, bn1_g, bn1_b, fc_W.T)


# speedup vs baseline: 1.0666x; 1.0666x over previous
import functools

import jax
import jax.numpy as jnp
import numpy as np
from jax import lax
from jax.experimental import pallas as pl
from jax.experimental.pallas import tpu as pltpu
from jax.experimental.pallas import tpu_sc as plsc

_N = 10000
_E = 320000
_D = 128

_NC = 2
_NS = 16
_CHUNK = 80
_CPC = _E // _CHUNK // _NC
_TPT = _CPC // _NS
_PAIRS = (_TPT - 1) // 2
_RPT = 624

_BE = 1280


@functools.partial(
    pl.kernel,
    out_type=jax.ShapeDtypeStruct((_NC, _N, _D), jnp.float32),
    mesh=plsc.VectorSubcoreMesh(core_axis_name="c", subcore_axis_name="s"),
    compiler_params=pltpu.CompilerParams(needs_layout_passes=False),
    scratch_types=[
        pltpu.VMEM((_CHUNK,), jnp.int32),
        pltpu.VMEM((_CHUNK,), jnp.int32),
        pltpu.VMEM((_CHUNK,), jnp.int32),
        pltpu.VMEM((_CHUNK,), jnp.int32),
        pltpu.VMEM((_CHUNK, _D), jnp.float32),
        pltpu.VMEM((_CHUNK, _D), jnp.float32),
        pltpu.VMEM((_CHUNK, _D), jnp.float32),
        pltpu.VMEM((_CHUNK, _D), jnp.float32),
        pltpu.SemaphoreType.DMA,
        pltpu.SemaphoreType.DMA,
        pltpu.SemaphoreType.DMA,
        pltpu.SemaphoreType.DMA,
        pltpu.SemaphoreType.DMA,
        pltpu.SemaphoreType.DMA,
        pltpu.SemaphoreType.DMA,
        pltpu.SemaphoreType.DMA,
        pltpu.VMEM_SHARED((_N, _D), jnp.float32),
    ],
)
def _sc_aggregate(h_hbm, ea_hbm, src_hbm, dst_hbm, out_hbm,
                  src0, src1, dst0, dst1, rows0, rows1, eab0, eab1,
                  si0, si1, sg0, sg1, se0, se1, ss0, ss1, acc):
    c = lax.axis_index("c")
    s = lax.axis_index("s")
    srcv = (src0, src1)
    dstv = (dst0, dst1)
    rows = (rows0, rows1)
    eab = (eab0, eab1)
    si = (si0, si1)
    sg = (sg0, sg1)
    se = (se0, se1)
    ss = (ss0, ss1)
    zero = jnp.zeros((16,), jnp.float32)

    def _zrow(r, carry):
        for j in range(8):
            rows0[r, pl.ds(j * 16, 16)] = zero
        return carry
    lax.fori_loop(0, _CHUNK, _zrow, 0)

    base = s * _RPT
    for j in range(_RPT // _CHUNK):
        pltpu.sync_copy(rows0, acc.at[pl.ds(base + j * _CHUNK, _CHUNK)])
    _REM = _RPT - (_RPT // _CHUNK) * _CHUNK
    pltpu.sync_copy(rows0.at[pl.ds(0, _REM)],
                    acc.at[pl.ds(base + _RPT - _REM, _REM)])

    @pl.when(s == _NS - 1)
    def _ztail():
        pltpu.sync_copy(rows0.at[pl.ds(0, _N - _NS * _RPT)],
                        acc.at[pl.ds(_NS * _RPT, _N - _NS * _RPT)])
    plsc.subcore_barrier()

    def _echunk(t):
        return (c * _CPC + t * _NS + s) * _CHUNK

    def _issue_idx(t, b):
        e0 = _echunk(t)
        pltpu.async_copy(src_hbm.at[pl.ds(e0, _CHUNK)], srcv[b], si[b])
        pltpu.async_copy(dst_hbm.at[pl.ds(e0, _CHUNK)], dstv[b], si[b])

    def _drain_idx(b):
        pltpu.make_async_copy(src_hbm.at[pl.ds(0, _CHUNK)], srcv[b], si[b]).wait()
        pltpu.make_async_copy(dst_hbm.at[pl.ds(0, _CHUNK)], dstv[b], si[b]).wait()

    def _issue_dat(t, b):
        pltpu.async_copy(h_hbm.at[srcv[b]], rows[b], sg[b])
        pltpu.async_copy(ea_hbm.at[pl.ds(_echunk(t), _CHUNK)], eab[b], se[b])

    def _drain_dat(b):
        pltpu.make_async_copy(h_hbm.at[pl.ds(0, _CHUNK)], rows[b], sg[b]).wait()
        pltpu.make_async_copy(ea_hbm.at[pl.ds(0, _CHUNK)], eab[b], se[b]).wait()

    def _compute(b):
        def _qrow(q, carry2):
            for k in range(8):
                sl = pl.ds(16 * k, 16)
                eab[b][q, sl] = jnp.maximum(rows[b][q, sl] + eab[b][q, sl], 0.0)
            return carry2
        lax.fori_loop(0, _CHUNK, _qrow, 0)

    def _issue_scat(b):
        pltpu.async_copy(eab[b], acc.at[dstv[b]], ss[b], add=True)

    def _drain_scat(b):
        pltpu.make_async_copy(ea_hbm.at[pl.ds(0, _CHUNK)], eab[b], ss[b]).wait()

    _issue_idx(0, 0)
    _drain_idx(0)
    _issue_dat(0, 0)
    _issue_idx(1, 1)

    def _pair(j, carry):
        for b in (0, 1):
            t = 2 * j + b
            nb = 1 - b
            _drain_idx(nb)
            if b == 0:
                @pl.when(j > 0)
                def _():
                    _drain_scat(nb)
            else:
                _drain_scat(nb)
            _issue_dat(t + 1, nb)
            _drain_dat(b)
            _compute(b)
            _issue_scat(b)
            if b == 0:
                _issue_idx(t + 2, b)
            else:
                @pl.when(j < _PAIRS - 1)
                def _():
                    _issue_idx(t + 2, b)
        return carry
    lax.fori_loop(0, _PAIRS, _pair, 0)

    _drain_scat(1)
    _drain_dat(0)
    _compute(0)
    pltpu.sync_copy(eab0, acc.at[dstv[0]], add=True)
    plsc.subcore_barrier()

    pltpu.sync_copy(acc.at[pl.ds(base, _RPT)],
                    out_hbm.at[c, pl.ds(base, _RPT)])

    @pl.when(s == _NS - 1)
    def _dtail():
        pltpu.sync_copy(acc.at[pl.ds(_NS * _RPT, _N - _NS * _RPT)],
                        out_hbm.at[c, pl.ds(_NS * _RPT, _N - _NS * _RPT)])


def _edge_mm_body(a_ref, w_ref, b_ref, o_ref):
    a = a_ref[...]
    o_ref[...] = jnp.dot(a, w_ref[...], preferred_element_type=jnp.float32) + b_ref[...]


def _edge_mm(edge_attr, wt, b):
    grid = (_E // _BE,)
    blk = pl.BlockSpec((_BE, _D), lambda i: (i, 0))
    wblk = pl.BlockSpec((_D, _D), lambda i: (0, 0))
    bblk = pl.BlockSpec((1, _D), lambda i: (0, 0))
    return pl.pallas_call(
        _edge_mm_body,
        grid=grid,
        in_specs=[blk, wblk, bblk],
        out_specs=blk,
        out_shape=jax.ShapeDtypeStruct((_E, _D), jnp.float32),
    )(edge_attr, wt, b)


def _bn(x, g, b):
    m = jnp.mean(x, axis=0, keepdims=True)
    xc = x - m
    v = jnp.mean(xc * xc, axis=0, keepdims=True)
    return xc * lax.rsqrt(v + 1e-5) * g + b


def _bn_in_body(x_ref, g_ref, b_ref, o_ref):
    o_ref[...] = _bn(x_ref[...], g_ref[...], b_ref[...])


def _bn_in(x, g, b):
    return pl.pallas_call(
        _bn_in_body,
        out_shape=jax.ShapeDtypeStruct((_N, _D), jnp.float32),
    )(x, g.reshape(1, _D), b.reshape(1, _D))


def _node0_body(h_ref, p0_ref, p1_ref, w_ref, b_ref, g_ref, bb_ref, o_ref):
    u = h_ref[...] + p0_ref[...] + p1_ref[...]
    t = jnp.tanh(jnp.dot(u, w_ref[...], preferred_element_type=jnp.float32) + b_ref[...])
    o_ref[...] = _bn(t, g_ref[...], bb_ref[...])


def _node0(h, p0, p1, wt, b, g, bb):
    return pl.pallas_call(
        _node0_body,
        out_shape=jax.ShapeDtypeStruct((_N, _D), jnp.float32),
    )(h, p0, p1, wt, b.reshape(1, _D), g.reshape(1, _D), bb.reshape(1, _D))


def _node1_body(h1_ref, p0_ref, p1_ref, w_ref, b_ref, g_ref, bb_ref, fc_ref, o_ref):
    h1 = h1_ref[...]
    u = h1 + p0_ref[...] + p1_ref[...]
    t = jnp.tanh(jnp.dot(u, w_ref[...], preferred_element_type=jnp.float32) + b_ref[...])
    h2 = _bn(t, g_ref[...], bb_ref[...])
    h3 = jnp.tanh(jnp.dot(h2, fc_ref[...], preferred_element_type=jnp.float32))
    o_ref[:, 0:_D] = h1
    o_ref[:, _D:2 * _D] = h2
    o_ref[:, 2 * _D:3 * _D] = h3


def _node1(h1, p0, p1, wt, b, g, bb, fct):
    return pl.pallas_call(
        _node1_body,
        out_shape=jax.ShapeDtypeStruct((_N, 3 * _D), jnp.float32),
    )(h1, p0, p1, wt, b.reshape(1, _D), g.reshape(1, _D), bb.reshape(1, _D), fct)


def kernel(x, edge_index, edge_attr, bn_in_g, bn_in_b,
           lin_e0_W, lin_e0_b, nn0_W, nn0_b, bn0_g, bn0_b,
           lin_e1_W, lin_e1_b, nn1_W, nn1_b, bn1_g, bn1_b,
           fc_W):
    ea0 = _edge_mm(edge_attr, lin_e0_W.T, lin_e0_b.reshape(1, _D))
    ea1 = _edge_mm(edge_attr, lin_e1_W.T, lin_e1_b.reshape(1, _D))
    h = _bn_in(x, bn_in_g, bn_in_b)
    src = edge_index[0]
    dst = edge_index[1]

    parts0 = _sc_aggregate(h, ea0, src, dst)
    h1 = _node0(h, parts0[0], parts0[1], nn0_W.T, nn0_b, bn0_g, bn0_b)

    parts1 = _sc_aggregate(h1, ea1, src, dst)
    return _node1(h1, parts1[0], parts1[1], nn1_W.T, nn1_b, bn1_g, bn1_b, fc_W.T)
